# baseline (speedup 1.0000x reference)
import jax
import jax.numpy as jnp
from jax import lax
from jax.experimental import pallas as pl
from jax.experimental.pallas import tpu as pltpu
from jax.experimental.pallas import tpu_sc as plsc

N_NODES = 10000
D = 128
N_EDGES = 320000

NC = 2
NS = 16
NW = NC * NS

CHUNK = 128
NBUF = 2
CHUNKS = 80
NGROUP = CHUNKS // NBUF
EPW = CHUNKS * CHUNK
EPAD = EPW * NW

NPAD = 10240
ROWS_PER_TILE = NPAD // NS
BLK = 256


def _sc_agg_body(x_hbm, eidx_hbm, out_hbm, ebuf, rows_v, gsems, ssems, isems,
                 agg_sh):
    c = lax.axis_index("c")
    s = lax.axis_index("s")
    wid = s * NC + c

    z = jnp.zeros((16,), jnp.float32)

    def zrow(i, carry):
        for j in range(8):
            rows_v[0, i, pl.ds(j * 16, 16)] = z
        return carry

    lax.fori_loop(0, CHUNK, zrow, 0)
    for k in range(ROWS_PER_TILE // CHUNK):
        pltpu.sync_copy(rows_v.at[0],
                        agg_sh.at[pl.ds(s * ROWS_PER_TILE + k * CHUNK, CHUNK)])
    plsc.subcore_barrier()

    def idx_load(g, p):
        pltpu.async_copy(eidx_hbm.at[wid, g], ebuf.at[p], isems.at[p])

    def idx_wait(p):
        pltpu.make_async_copy(eidx_hbm.at[wid, 0], ebuf.at[p],
                              isems.at[p]).wait()

    def gather(p, b):
        pltpu.async_copy(x_hbm.at[ebuf.at[p, 0, b]], rows_v.at[b],
                         gsems.at[b])

    def gather_wait(b):
        pltpu.make_async_copy(x_hbm.at[ebuf.at[0, 0, 0]], rows_v.at[b],
                              gsems.at[b]).wait()

    def scatter(p, b):
        pltpu.async_copy(rows_v.at[b], agg_sh.at[ebuf.at[p, 1, b]],
                         ssems.at[b], add=True)

    def scatter_wait(b):
        pltpu.make_async_copy(rows_v.at[b], agg_sh.at[ebuf.at[0, 1, 0]],
                              ssems.at[b]).wait()

    idx_load(0, 0)
    idx_load(1, 1)
    idx_wait(0)
    for b in range(NBUF):
        gather(0, b)

    def pair(k, carry):
        for p in range(2):
            g = 2 * k + p
            for b in range(NBUF):
                gather_wait(b)
                scatter(p, b)
            for b in range(NBUF):
                scatter_wait(b)
            if p == 0:
                idx_wait(1)
                for b in range(NBUF):
                    gather(1, b)

                @pl.when(k < NGROUP // 2 - 1)
                def _():
                    idx_load(g + 2, 0)
            else:
                @pl.when(k < NGROUP // 2 - 1)
                def _():
                    idx_wait(0)
                    for b in range(NBUF):
                        gather(0, b)
                    idx_load(g + 2, 1)
        return carry

    lax.fori_loop(0, NGROUP // 2, pair, 0)
    plsc.subcore_barrier()

    pltpu.sync_copy(agg_sh.at[pl.ds(s * ROWS_PER_TILE, ROWS_PER_TILE)],
                    out_hbm.at[c, pl.ds(s * ROWS_PER_TILE, ROWS_PER_TILE)])


_sc_agg = pl.kernel(
    _sc_agg_body,
    out_type=jax.ShapeDtypeStruct((NC, NPAD, D), jnp.float32),
    mesh=plsc.VectorSubcoreMesh(core_axis_name="c", subcore_axis_name="s"),
    scratch_types=[
        pltpu.VMEM((2, 2, NBUF, CHUNK), jnp.int32),
        pltpu.VMEM((NBUF, CHUNK, D), jnp.float32),
        pltpu.SemaphoreType.DMA((NBUF,)),
        pltpu.SemaphoreType.DMA((NBUF,)),
        pltpu.SemaphoreType.DMA((2,)),
        pltpu.VMEM_SHARED((NPAD, D), jnp.float32),
    ],
)


def _tc_mlp_body(x_ref, agg_ref, wa_ref, ba_ref, wb_ref, bb_ref, o_ref):
    h = x_ref[...] + agg_ref[0] + agg_ref[1]
    h = jnp.maximum(jnp.dot(h, wa_ref[...],
                            preferred_element_type=jnp.float32) + ba_ref[...],
                    0.0)
    h = jnp.maximum(jnp.dot(h, wb_ref[...],
                            preferred_element_type=jnp.float32) + bb_ref[...],
                    0.0)
    rows = (pl.program_id(0) * BLK
            + lax.broadcasted_iota(jnp.int32, (BLK, 1), 0))
    o_ref[...] = jnp.where(rows < N_NODES, h, 0.0)


_tc_mlp = pl.pallas_call(
    _tc_mlp_body,
    grid=(NPAD // BLK,),
    in_specs=[
        pl.BlockSpec((BLK, D), lambda i: (i, 0)),
        pl.BlockSpec((NC, BLK, D), lambda i: (0, i, 0)),
        pl.BlockSpec((D, D), lambda i: (0, 0)),
        pl.BlockSpec((1, D), lambda i: (0, 0)),
        pl.BlockSpec((D, D), lambda i: (0, 0)),
        pl.BlockSpec((1, D), lambda i: (0, 0)),
    ],
    out_specs=pl.BlockSpec((BLK, D), lambda i: (i, 0)),
    out_shape=jax.ShapeDtypeStruct((NPAD, D), jnp.float32),
)


@jax.jit
def kernel(x, edge_index, W1a, b1a, W1b, b1b, W2a, b2a, W2b, b2b):
    src = edge_index[0].astype(jnp.int32)
    dst = edge_index[1].astype(jnp.int32)
    pad = jnp.full((EPAD - N_EDGES,), N_NODES, jnp.int32)
    src = jnp.concatenate([src, pad]).reshape(NW, NGROUP, 1, NBUF, CHUNK)
    dst = jnp.concatenate([dst, pad]).reshape(NW, NGROUP, 1, NBUF, CHUNK)
    eidx = jnp.concatenate([src, dst], axis=2)
    x_pad = jnp.zeros((NPAD, D), jnp.float32).at[:N_NODES].set(x)

    agg1 = _sc_agg(x_pad, eidx)
    h1 = _tc_mlp(x_pad, agg1, W1a, b1a.reshape(1, D), W1b, b1b.reshape(1, D))
    agg2 = _sc_agg(h1, eidx)
    out = _tc_mlp(h1, agg2, W2a, b2a.reshape(1, D), W2b, b2b.reshape(1, D))
    return out[:N_NODES]
